# baseline (speedup 1.0000x reference)
import functools

import jax
import jax.numpy as jnp
from jax import lax
from jax.experimental import pallas as pl
from jax.experimental.pallas import tpu as pltpu
from jax.experimental.pallas import tpu_sc as plsc

NUM_B = 4096
NUM_T = 200
NUM_TOKENS = NUM_B * NUM_T
DIM = 64
NUM_WORKERS = 32
PER_WORKER = NUM_TOKENS // NUM_WORKERS
CHUNK = 128
NUM_CHUNKS = PER_WORKER // CHUNK
NBUF = 4
SUPER = NUM_CHUNKS // (2 * NBUF)


def _embedding_gather_call():
    mesh = plsc.VectorSubcoreMesh(core_axis_name="c", subcore_axis_name="s")

    @functools.partial(
        pl.kernel,
        mesh=mesh,
        out_type=jax.ShapeDtypeStruct((NUM_TOKENS, DIM), jnp.float32),
        compiler_params=pltpu.CompilerParams(use_tc_tiling_on_sc=False),
        scratch_types=(
            [pltpu.VMEM((NUM_CHUNKS, CHUNK), jnp.int32)]
            + [pltpu.VMEM((CHUNK, DIM), jnp.float32) for _ in range(2 * NBUF)]
            + [pltpu.SemaphoreType.DMA for _ in range(4)]
        ),
    )
    def gather_kernel(idx_hbm, table_hbm, out_hbm, idx_all, *bufs_and_sems):
        rows = bufs_and_sems[: 2 * NBUF]
        gsem_a, ssem_a, gsem_b, ssem_b = bufs_and_sems[2 * NBUF:]
        rows_a, rows_b = rows[:NBUF], rows[NBUF:]

        wid = lax.axis_index("s") * 2 + lax.axis_index("c")
        base = wid * PER_WORKER
        crow = wid * NUM_CHUNKS

        pltpu.sync_copy(idx_hbm.at[pl.ds(crow, NUM_CHUNKS)], idx_all)

        def start_gathers(ci, bank_rows, gsem):
            return [
                pltpu.async_copy(
                    table_hbm.at[idx_all.at[ci + b]], bank_rows[b], gsem)
                for b in range(NBUF)
            ]

        def start_stores(ci, bank_rows, ssem):
            for b in range(NBUF):
                off = base + (ci + b) * CHUNK
                pltpu.async_copy(
                    bank_rows[b], out_hbm.at[pl.ds(off, CHUNK)], ssem)

        def drain_stores(bank_rows, ssem):
            for b in range(NBUF):
                pltpu.make_async_copy(
                    bank_rows[b], out_hbm.at[pl.ds(base, CHUNK)], ssem).wait()

        def body(s, carry):
            ci_a = s * 2 * NBUF
            ci_b = ci_a + NBUF

            @pl.when(s > 0)
            def _():
                drain_stores(rows_a, ssem_a)

            ga = start_gathers(ci_a, rows_a, gsem_a)

            @pl.when(s > 0)
            def _():
                drain_stores(rows_b, ssem_b)

            for cp in ga:
                cp.wait()
            start_stores(ci_a, rows_a, ssem_a)

            gb = start_gathers(ci_b, rows_b, gsem_b)
            for cp in gb:
                cp.wait()
            start_stores(ci_b, rows_b, ssem_b)
            return carry

        lax.fori_loop(0, SUPER, body, 0)
        drain_stores(rows_a, ssem_a)
        drain_stores(rows_b, ssem_b)

    return gather_kernel


_gather = _embedding_gather_call()


def kernel(token_ids, weight):
    flat = token_ids.T.reshape(NUM_TOKENS // CHUNK, CHUNK).astype(jnp.int32)
    out = _gather(flat, weight)
    return out.reshape(NUM_T, NUM_B, DIM).transpose(1, 0, 2)


# speedup vs baseline: 1.3408x; 1.3408x over previous
import functools

import jax
import jax.numpy as jnp
from jax import lax
from jax.experimental import pallas as pl
from jax.experimental.pallas import tpu as pltpu
from jax.experimental.pallas import tpu_sc as plsc

NUM_B = 4096
NUM_T = 200
NUM_TOKENS = NUM_B * NUM_T
DIM = 64
NUM_WORKERS = 32
PER_WORKER = NUM_TOKENS // NUM_WORKERS
CHUNK = 128
NUM_CHUNKS = PER_WORKER // CHUNK
NBUF = 4
SUPER = NUM_CHUNKS // (2 * NBUF)


def _embedding_gather_call():
    mesh = plsc.VectorSubcoreMesh(core_axis_name="c", subcore_axis_name="s")

    @functools.partial(
        pl.kernel,
        mesh=mesh,
        out_type=jax.ShapeDtypeStruct((NUM_TOKENS, 2 * DIM), jnp.float32),
        compiler_params=pltpu.CompilerParams(use_tc_tiling_on_sc=False),
        scratch_types=(
            [pltpu.VMEM((NUM_CHUNKS, CHUNK), jnp.int32)]
            + [pltpu.VMEM((CHUNK, DIM), jnp.float32) for _ in range(2 * NBUF)]
            + [pltpu.SemaphoreType.DMA for _ in range(4)]
        ),
    )
    def gather_kernel(idx_hbm, table_hbm, out_hbm, idx_all, *bufs_and_sems):
        rows = bufs_and_sems[: 2 * NBUF]
        gsem_a, ssem_a, gsem_b, ssem_b = bufs_and_sems[2 * NBUF:]
        rows_a, rows_b = rows[:NBUF], rows[NBUF:]

        wid = lax.axis_index("s") * 2 + lax.axis_index("c")
        base = wid * PER_WORKER
        crow = wid * NUM_CHUNKS

        pltpu.sync_copy(idx_hbm.at[pl.ds(crow, NUM_CHUNKS)], idx_all)

        def start_gathers(ci, bank_rows, gsem):
            return [
                pltpu.async_copy(
                    table_hbm.at[idx_all.at[ci + b]], bank_rows[b], gsem)
                for b in range(NBUF)
            ]

        def start_stores(ci, bank_rows, ssem):
            for b in range(NBUF):
                off = base + (ci + b) * CHUNK
                pltpu.async_copy(
                    bank_rows[b],
                    out_hbm.at[pl.ds(off, CHUNK), pl.ds(0, DIM)], ssem)

        def drain_stores(bank_rows, ssem):
            for b in range(NBUF):
                pltpu.make_async_copy(
                    bank_rows[b],
                    out_hbm.at[pl.ds(base, CHUNK), pl.ds(0, DIM)], ssem).wait()

        def body(s, carry):
            ci_a = s * 2 * NBUF
            ci_b = ci_a + NBUF

            @pl.when(s > 0)
            def _():
                drain_stores(rows_a, ssem_a)

            ga = start_gathers(ci_a, rows_a, gsem_a)

            @pl.when(s > 0)
            def _():
                drain_stores(rows_b, ssem_b)

            for cp in ga:
                cp.wait()
            start_stores(ci_a, rows_a, ssem_a)

            gb = start_gathers(ci_b, rows_b, gsem_b)
            for cp in gb:
                cp.wait()
            start_stores(ci_b, rows_b, ssem_b)
            return carry

        lax.fori_loop(0, SUPER, body, 0)
        drain_stores(rows_a, ssem_a)
        drain_stores(rows_b, ssem_b)

    return gather_kernel


_gather = _embedding_gather_call()


def kernel(token_ids, weight):
    flat = token_ids.T.reshape(NUM_TOKENS // CHUNK, CHUNK).astype(jnp.int32)
    out = _gather(flat, weight)
    return out.reshape(NUM_T, NUM_B, 2 * DIM)[:, :, :DIM].transpose(1, 0, 2)
